# baseline (speedup 1.0000x reference)
import functools

import jax
import jax.numpy as jnp
from jax import lax
from jax.experimental import pallas as pl
from jax.experimental.pallas import tpu as pltpu
from jax.experimental.pallas import tpu_sc as plsc

NUM_EMB = 1_000_000
DIM = 64
BATCH = 16384
HIST = 50

NC = 2
NS = 16
NW = NC * NS

BTILE = 128
NBB = BATCH // BTILE
NBLOCKS = HIST * NBB
PER_W = NBLOCKS // NW


def _mesh():
    return plsc.VectorSubcoreMesh(core_axis_name="c", subcore_axis_name="s")


@functools.partial(
    pl.kernel,
    out_type=jax.ShapeDtypeStruct((HIST, DIM // 8, NBB, 8, BTILE),
                                  jnp.float32),
    mesh=_mesh(),
    compiler_params=pltpu.CompilerParams(use_tc_tiling_on_sc=False,
                                         needs_layout_passes=False),
    scratch_types=[
        pltpu.VMEM((PER_W, BTILE), jnp.int32),
        pltpu.VMEM((BTILE, DIM), jnp.float32),
        pltpu.VMEM((BTILE, DIM), jnp.float32),
        pltpu.VMEM((DIM // 8, 8, BTILE), jnp.float32),
        pltpu.VMEM((DIM // 8, 8, BTILE), jnp.float32),
        pltpu.SemaphoreType.DMA,
        pltpu.SemaphoreType.DMA,
        pltpu.SemaphoreType.DMA,
        pltpu.SemaphoreType.DMA,
    ],
)
def _gather_kernel(idx_hbm, table_hbm, out_hbm, idx_all, rows0, rows1,
                   t0, t1, g0sem, g1sem, o0sem, o1sem):
    wid = lax.axis_index("s") * NC + lax.axis_index("c")
    base = wid * PER_W
    pltpu.sync_copy(idx_hbm.at[pl.ds(base, PER_W)], idx_all)

    rows = (rows0, rows1)
    tbuf = (t0, t1)
    gsem = (g0sem, g1sem)
    osem = (o0sem, o1sem)

    i16 = lax.iota(jnp.int32, 16)
    zeros16 = jnp.zeros((16,), jnp.int32)
    clvec = i16 % 8
    chbase = i16 // 8
    chvecs = [chbase + (c0 // 8) for c0 in (0, 16, 32, 48)]

    def fire_gather(i, slot):
        pltpu.async_copy(table_hbm.at[idx_all.at[i]], rows[slot], gsem[slot])

    def drain(sem, slot):
        pltpu.make_async_copy(table_hbm.at[pl.ds(0, BTILE)], rows[slot],
                              sem).wait()

    def transpose(slot):
        r = rows[slot]
        t = tbuf[slot]

        for b in range(BTILE):
            bvec = zeros16 + b
            for ci, c0 in enumerate((0, 16, 32, 48)):
                v = r[b, pl.ds(c0, 16)]
                plsc.store_scatter(t, [chvecs[ci], clvec, bvec], v)

    def fire_out(g, slot):
        th = g // NBB
        bb = g % NBB
        pltpu.async_copy(tbuf[slot], out_hbm.at[th, :, bb, :, :], osem[slot])

    fire_gather(0, 0)
    fire_gather(1, 1)

    @pl.loop(0, PER_W, step=2)
    def _(i):
        for s in (0, 1):
            ii = i + s
            drain(gsem[s], s)

            @pl.when(ii >= 2)
            def _():
                drain(osem[s], s)

            transpose(s)
            fire_out(base + ii, s)

            @pl.when(ii + 2 < PER_W)
            def _():
                fire_gather(ii + 2, s)

    drain(osem[0], 0)
    drain(osem[1], 1)


def kernel(token_ids, W):
    idx = token_ids.astype(jnp.int32).T.reshape(NBLOCKS, BTILE)
    x = _gather_kernel(idx, W)
    return x.transpose(2, 4, 0, 1, 3).reshape(BATCH, HIST, DIM)


# speedup vs baseline: 1.3542x; 1.3542x over previous
import functools

import jax
import jax.numpy as jnp
from jax import lax
from jax.experimental import pallas as pl
from jax.experimental.pallas import tpu as pltpu
from jax.experimental.pallas import tpu_sc as plsc

NUM_EMB = 1_000_000
DIM = 64
BATCH = 16384
HIST = 50

NC = 2
NS = 16
NW = NC * NS

BTILE = 128
NBB = BATCH // BTILE
NBLOCKS = HIST * NBB
PER_W = NBLOCKS // NW


def _mesh():
    return plsc.VectorSubcoreMesh(core_axis_name="c", subcore_axis_name="s")


@functools.partial(
    pl.kernel,
    out_type=jax.ShapeDtypeStruct((HIST, DIM // 8, NBB, 8, BTILE),
                                  jnp.float32),
    mesh=_mesh(),
    compiler_params=pltpu.CompilerParams(use_tc_tiling_on_sc=False,
                                         needs_layout_passes=False),
    scratch_types=[
        pltpu.VMEM((PER_W, BTILE), jnp.int32),
        pltpu.VMEM((BTILE, DIM), jnp.float32),
        pltpu.VMEM((BTILE, DIM), jnp.float32),
        pltpu.VMEM((DIM // 8, 8, BTILE), jnp.float32),
        pltpu.VMEM((DIM // 8, 8, BTILE), jnp.float32),
        pltpu.SemaphoreType.DMA,
        pltpu.SemaphoreType.DMA,
        pltpu.SemaphoreType.DMA,
        pltpu.SemaphoreType.DMA,
    ],
)
def _gather_kernel(idx_hbm, table_hbm, out_hbm, idx_all, rows0, rows1,
                   t0, t1, g0sem, g1sem, o0sem, o1sem):
    wid = lax.axis_index("s") * NC + lax.axis_index("c")
    base = wid * PER_W
    pltpu.sync_copy(idx_hbm.at[pl.ds(base, PER_W)], idx_all)

    rows = (rows0, rows1)
    tbuf = (t0, t1)
    gsem = (g0sem, g1sem)
    osem = (o0sem, o1sem)

    i16 = lax.iota(jnp.int32, 16)
    perm = [(i16 + d) & 15 for d in range(16)]

    def fire_gather(i, slot):
        pltpu.async_copy(table_hbm.at[idx_all.at[i]], rows[slot], gsem[slot])

    def drain(sem, slot):
        pltpu.make_async_copy(table_hbm.at[pl.ds(0, BTILE)], rows[slot],
                              sem).wait()

    def transpose(slot):
        r = rows[slot]
        t = tbuf[slot]

        @pl.loop(0, BTILE // 16)
        def _(bblk):
            bvec = i16 + bblk * 16
            for cb in range(DIM // 16):
                for d in range(16):
                    cvec = perm[d] + (cb * 16)
                    v = plsc.load_gather(r, [bvec, cvec])
                    plsc.store_scatter(
                        t, [cvec >> 3, cvec & 7, bvec], v)

    def fire_out(g, slot):
        th = g // NBB
        bb = g % NBB
        pltpu.async_copy(tbuf[slot], out_hbm.at[th, :, bb, :, :], osem[slot])

    fire_gather(0, 0)
    fire_gather(1, 1)

    @pl.loop(0, PER_W, step=2)
    def _(i):
        for s in (0, 1):
            ii = i + s
            drain(gsem[s], s)

            @pl.when(ii >= 2)
            def _():
                drain(osem[s], s)

            transpose(s)
            fire_out(base + ii, s)

            @pl.when(ii + 2 < PER_W)
            def _():
                fire_gather(ii + 2, s)

    drain(osem[0], 0)
    drain(osem[1], 1)


def kernel(token_ids, W):
    idx = token_ids.astype(jnp.int32).T.reshape(NBLOCKS, BTILE)
    x = _gather_kernel(idx, W)
    return x.transpose(2, 4, 0, 1, 3).reshape(BATCH, HIST, DIM)


# speedup vs baseline: 1.6465x; 1.2158x over previous
import functools

import jax
import jax.numpy as jnp
from jax import lax
from jax.experimental import pallas as pl
from jax.experimental.pallas import tpu as pltpu
from jax.experimental.pallas import tpu_sc as plsc

NUM_EMB = 1_000_000
DIM = 64
BATCH = 16384
HIST = 50

NC = 2
NS = 16
NW = NC * NS

BTILE = 128
NBB = BATCH // BTILE
NBLOCKS = HIST * NBB
PER_W = NBLOCKS // NW


def _mesh():
    return plsc.VectorSubcoreMesh(core_axis_name="c", subcore_axis_name="s")


@functools.partial(
    pl.kernel,
    out_type=jax.ShapeDtypeStruct((HIST, DIM // 8, NBB, 8, BTILE),
                                  jnp.float32),
    mesh=_mesh(),
    compiler_params=pltpu.CompilerParams(use_tc_tiling_on_sc=False,
                                         needs_layout_passes=False),
    scratch_types=[
        pltpu.VMEM((PER_W, BTILE), jnp.int32),
        pltpu.VMEM((BTILE, DIM), jnp.float32),
        pltpu.VMEM((BTILE, DIM), jnp.float32),
        pltpu.VMEM((DIM // 8, 8, BTILE), jnp.float32),
        pltpu.VMEM((DIM // 8, 8, BTILE), jnp.float32),
        pltpu.SemaphoreType.DMA,
        pltpu.SemaphoreType.DMA,
        pltpu.SemaphoreType.DMA,
        pltpu.SemaphoreType.DMA,
    ],
)
def _gather_kernel(idx_hbm, table_hbm, out_hbm, idx_all, rows0, rows1,
                   t0, t1, g0sem, g1sem, o0sem, o1sem):
    wid = lax.axis_index("s") * NC + lax.axis_index("c")
    base = wid * PER_W
    pltpu.sync_copy(idx_hbm.at[pl.ds(base, PER_W)], idx_all)

    rows = (rows0, rows1)
    tbuf = (t0, t1)
    gsem = (g0sem, g1sem)
    osem = (o0sem, o1sem)

    i16 = lax.iota(jnp.int32, 16)
    perm = [(i16 + d) & 15 for d in range(16)]

    def fire_gather(i, slot):
        pltpu.async_copy(table_hbm.at[idx_all.at[i]], rows[slot], gsem[slot])

    def drain(sem, slot):
        pltpu.make_async_copy(table_hbm.at[pl.ds(0, BTILE)], rows[slot],
                              sem).wait()

    def transpose(slot):
        r = rows[slot]
        t = tbuf[slot]

        @plsc.parallel_loop(0, BTILE // 16, unroll=2)
        def _(bblk):
            bvec = i16 + bblk * 16
            for cb in range(DIM // 16):
                for d in range(16):
                    cvec = perm[d] + (cb * 16)
                    v = plsc.load_gather(r, [bvec, cvec])
                    plsc.store_scatter(
                        t, [cvec >> 3, cvec & 7, bvec], v)

    def fire_out(g, slot):
        th = g // NBB
        bb = g % NBB
        pltpu.async_copy(tbuf[slot], out_hbm.at[th, :, bb, :, :], osem[slot])

    fire_gather(0, 0)
    fire_gather(1, 1)

    @pl.loop(0, PER_W, step=2)
    def _(i):
        for s in (0, 1):
            ii = i + s
            drain(gsem[s], s)

            @pl.when(ii >= 2)
            def _():
                drain(osem[s], s)

            transpose(s)
            fire_out(base + ii, s)

            @pl.when(ii + 2 < PER_W)
            def _():
                fire_gather(ii + 2, s)

    drain(osem[0], 0)
    drain(osem[1], 1)


def kernel(token_ids, W):
    idx = token_ids.astype(jnp.int32).T.reshape(NBLOCKS, BTILE)
    x = _gather_kernel(idx, W)
    return x.transpose(2, 4, 0, 1, 3).reshape(BATCH, HIST, DIM)
